# baseline (speedup 1.0000x reference)
import numpy as np
import jax
import jax.numpy as jnp
from jax import lax
from jax.experimental import pallas as pl
from jax.experimental.pallas import tpu as pltpu, tpu_sc as plsc

_TRI = (
    (1, 4, -1, -1), (1, -1, -1, -1), (1, 2, 5, -1), (2, -1, -1, -1),
    (2, 3, 6, -1), (3, -1, -1, -1), (3, 7, -1, -1), (4, -1, -1, -1),
    (1, 4, 5, 8), (5, -1, -1, -1), (2, 5, 6, 9), (6, -1, -1, -1),
    (3, 6, 7, 10), (7, -1, -1, -1), (4, 8, 11, -1), (8, -1, -1, -1),
    (5, 8, 9, 12), (9, -1, -1, -1), (6, 9, 10, 13), (10, -1, -1, -1),
    (7, 10, 14, -1), (11, -1, -1, -1), (8, 11, 12, 15), (12, -1, -1, -1),
    (9, 12, 13, 16), (13, -1, -1, -1), (10, 13, 14, 17), (14, -1, -1, -1),
    (11, 15, 18, -1), (15, -1, -1, -1), (12, 15, 16, 19), (16, -1, -1, -1),
    (13, 16, 17, 20), (17, -1, -1, -1), (14, 17, 21, -1), (18, -1, -1, -1),
    (15, 18, 19, 22), (19, -1, -1, -1), (16, 19, 20, 23), (20, -1, -1, -1),
    (17, 20, 21, 24), (21, -1, -1, -1), (18, 22, -1, -1), (22, -1, -1, -1),
    (19, 22, 23, -1), (23, -1, -1, -1), (20, 23, 24, -1), (24, -1, -1, -1),
    (21, 24, -1, -1),
)

_CELLS = tuple(tuple(v - 1 for v in row if v > 0) for row in _TRI)

_BC = 128
_NV = 24
_NCELL = 49
_L = 4096
_LC = 512
_NCH = _L // _LC
_NTASK = _BC * _NCH
_NW = 32
_TPW = _NTASK // _NW
_LANES = 16


def _body(x_hbm, out_hbm, xbuf, obuf, isem0, isem1, osem0, osem1):
    wid = lax.axis_index("s") * 2 + lax.axis_index("c")
    isems = (isem0, isem1)
    osems = (osem0, osem1)

    def task_coords(t):
        task = wid * _TPW + t
        return task // _NCH, (task % _NCH) * _LC

    def in_copy(t, s):
        bc, col = task_coords(t)
        return pltpu.make_async_copy(
            x_hbm.at[bc, :, pl.ds(col, _LC)], xbuf.at[s], isems[s])

    def out_copy(t, s):
        bc, col = task_coords(t)
        return pltpu.make_async_copy(
            obuf.at[s], out_hbm.at[bc, :, pl.ds(col, _LC)], osems[s])

    def compute(s):
        @plsc.parallel_loop(0, _LC // _LANES, unroll=2)
        def chunk(i):
            o = i * _LANES
            rows = [xbuf[s, v, pl.ds(o, _LANES)] for v in range(_NV)]
            for ci, vs in enumerate(_CELLS):
                acc = rows[vs[0]]
                for v in vs[1:]:
                    acc = acc + rows[v]
                if len(vs) > 1:
                    acc = acc * np.float32(1.0 / len(vs))
                obuf[s, ci, pl.ds(o, _LANES)] = acc

    in_copy(0, 0).start()
    in_copy(1, 1).start()

    def pair(k, carry):
        for s in (0, 1):
            t = 2 * k + s
            in_copy(t, s).wait()

            @pl.when(t >= 2)
            def _():
                out_copy(t - 2, s).wait()

            compute(s)
            out_copy(t, s).start()

            @pl.when(t + 2 < _TPW)
            def _():
                in_copy(t + 2, s).start()
        return carry

    lax.fori_loop(0, _TPW // 2, pair, 0)
    out_copy(_TPW - 2, 0).wait()
    out_copy(_TPW - 1, 1).wait()


def kernel(x, b, l):
    x2 = x.reshape(_BC, _NV, _L)
    out = pl.kernel(
        _body,
        out_type=jax.ShapeDtypeStruct((_BC, _NCELL, _L), jnp.float32),
        mesh=plsc.VectorSubcoreMesh(core_axis_name="c", subcore_axis_name="s"),
        scratch_types=[
            pltpu.VMEM((2, _NV, _LC), jnp.float32),
            pltpu.VMEM((2, _NCELL, _LC), jnp.float32),
            pltpu.SemaphoreType.DMA,
            pltpu.SemaphoreType.DMA,
            pltpu.SemaphoreType.DMA,
            pltpu.SemaphoreType.DMA,
        ],
        compiler_params=pltpu.CompilerParams(use_tc_tiling_on_sc=True),
    )(x2)
    return out.reshape(x.shape[0], 2, 7, 7, _L)


# speedup vs baseline: 1.4905x; 1.4905x over previous
import numpy as np
import jax
import jax.numpy as jnp
from jax import lax
from jax.experimental import pallas as pl
from jax.experimental.pallas import tpu as pltpu, tpu_sc as plsc

_TRI = (
    (1, 4, -1, -1), (1, -1, -1, -1), (1, 2, 5, -1), (2, -1, -1, -1),
    (2, 3, 6, -1), (3, -1, -1, -1), (3, 7, -1, -1), (4, -1, -1, -1),
    (1, 4, 5, 8), (5, -1, -1, -1), (2, 5, 6, 9), (6, -1, -1, -1),
    (3, 6, 7, 10), (7, -1, -1, -1), (4, 8, 11, -1), (8, -1, -1, -1),
    (5, 8, 9, 12), (9, -1, -1, -1), (6, 9, 10, 13), (10, -1, -1, -1),
    (7, 10, 14, -1), (11, -1, -1, -1), (8, 11, 12, 15), (12, -1, -1, -1),
    (9, 12, 13, 16), (13, -1, -1, -1), (10, 13, 14, 17), (14, -1, -1, -1),
    (11, 15, 18, -1), (15, -1, -1, -1), (12, 15, 16, 19), (16, -1, -1, -1),
    (13, 16, 17, 20), (17, -1, -1, -1), (14, 17, 21, -1), (18, -1, -1, -1),
    (15, 18, 19, 22), (19, -1, -1, -1), (16, 19, 20, 23), (20, -1, -1, -1),
    (17, 20, 21, 24), (21, -1, -1, -1), (18, 22, -1, -1), (22, -1, -1, -1),
    (19, 22, 23, -1), (23, -1, -1, -1), (20, 23, 24, -1), (24, -1, -1, -1),
    (21, 24, -1, -1),
)

_CELLS = tuple(tuple(v - 1 for v in row if v > 0) for row in _TRI)

_BC = 128
_NV = 24
_NCELL = 49
_L = 4096
_LC = 512
_NCH = _L // _LC
_NTASK = _BC * _NCH
_NW = 32
_TPW = _NTASK // _NW
_LANES = 16


def _body(x_hbm, out_hbm, xbuf, obuf, isem0, isem1, osem0, osem1):
    wid = lax.axis_index("s") * 2 + lax.axis_index("c")
    isems = (isem0, isem1)
    osems = (osem0, osem1)

    def task_coords(t):
        task = wid * _TPW + t
        bc = task // _NCH
        return bc // 2, bc % 2, (task % _NCH) * _LC

    def in_copy(t, s):
        b, ch, col = task_coords(t)
        return pltpu.make_async_copy(
            x_hbm.at[b, ch, :, pl.ds(col, _LC)], xbuf.at[s], isems[s])

    def out_copy(t, s):
        b, ch, col = task_coords(t)
        return pltpu.make_async_copy(
            obuf.at[s], out_hbm.at[b, ch, :, :, pl.ds(col, _LC)], osems[s])

    def compute(s):
        @plsc.parallel_loop(0, _LC // _LANES, unroll=2)
        def chunk(i):
            o = i * _LANES
            rows = [xbuf[s, v, pl.ds(o, _LANES)] for v in range(_NV)]
            for ci, vs in enumerate(_CELLS):
                acc = rows[vs[0]]
                for v in vs[1:]:
                    acc = acc + rows[v]
                if len(vs) > 1:
                    acc = acc * np.float32(1.0 / len(vs))
                obuf[s, ci // 7, ci % 7, pl.ds(o, _LANES)] = acc

    in_copy(0, 0).start()
    in_copy(1, 1).start()

    def pair(k, carry):
        for s in (0, 1):
            t = 2 * k + s
            in_copy(t, s).wait()

            @pl.when(t >= 2)
            def _():
                out_copy(t - 2, s).wait()

            compute(s)
            out_copy(t, s).start()

            @pl.when(t + 2 < _TPW)
            def _():
                in_copy(t + 2, s).start()
        return carry

    lax.fori_loop(0, _TPW // 2, pair, 0)
    out_copy(_TPW - 2, 0).wait()
    out_copy(_TPW - 1, 1).wait()


def kernel(x, b, l):
    out = pl.kernel(
        _body,
        out_type=jax.ShapeDtypeStruct((64, 2, 7, 8, _L), jnp.float32),
        mesh=plsc.VectorSubcoreMesh(core_axis_name="c", subcore_axis_name="s"),
        scratch_types=[
            pltpu.VMEM((2, _NV, _LC), jnp.float32),
            pltpu.VMEM((2, 7, 8, _LC), jnp.float32),
            pltpu.SemaphoreType.DMA,
            pltpu.SemaphoreType.DMA,
            pltpu.SemaphoreType.DMA,
            pltpu.SemaphoreType.DMA,
        ],
        compiler_params=pltpu.CompilerParams(use_tc_tiling_on_sc=True),
    )(x)
    return out[:, :, :, :7, :]
